# baseline (speedup 1.0000x reference)
import functools

import jax
import jax.numpy as jnp
from jax import lax
from jax.experimental import pallas as pl
from jax.experimental.pallas import tpu as pltpu
from jax.experimental.pallas import tpu_sc as plsc

NC = 2
NS = 16
NW = NC * NS
CHUNK = 128


def _pad_rows(n):
    per_tile = -(-n // (NS * CHUNK)) * CHUNK
    return NS * per_tile


def _mesh():
    return plsc.VectorSubcoreMesh(core_axis_name="c", subcore_axis_name="s")


def _zero_rows(buf, rows, width):
    @pl.loop(0, rows)
    def _(i):
        for j in range(width // 16):
            buf[i, pl.ds(j * 16, 16)] = jnp.zeros((16,), jnp.float32)


def _copy_rows_spmem(src_buf, dst_sh, base, rows):
    for b in range(rows // CHUNK):
        pltpu.sync_copy(src_buf, dst_sh.at[pl.ds(base + b * CHUNK, CHUNK)])


def _copy_out_spmem(src_sh, out_hbm, cid, base, rows):
    for b in range(rows // CHUNK):
        pltpu.sync_copy(src_sh.at[pl.ds(base + b * CHUNK, CHUNK)],
                        out_hbm.at[cid, pl.ds(base + b * CHUNK, CHUNK)])


def _edge_chunks(e):
    per_tile = -(-e // (NW * CHUNK))
    return per_tile + (per_tile % 2)


@functools.lru_cache(maxsize=None)
def _make_degree_kernel(n, e):
    nch = _edge_chunks(e)
    npad = _pad_rows(n)

    @functools.partial(
        pl.kernel,
        mesh=_mesh(),
        compiler_params=pltpu.CompilerParams(needs_layout_passes=False),
        out_type=(jax.ShapeDtypeStruct((NW * npad,), jnp.float32),
                  jax.ShapeDtypeStruct((NW * npad,), jnp.float32)),
        scratch_types=[
            pltpu.VMEM((nch, CHUNK), jnp.int32),
            pltpu.VMEM((nch, CHUNK), jnp.int32),
            pltpu.VMEM((npad,), jnp.float32),
            pltpu.VMEM((npad,), jnp.float32),
        ],
    )
    def k(src_hbm, dst_hbm, outs_hbm, outd_hbm,
          src_i, dst_i, cnts_v, cntd_v):
        cid = lax.axis_index("c")
        sid = lax.axis_index("s")
        wid = sid * NC + cid

        pltpu.sync_copy(src_hbm.at[pl.ds(wid * nch, nch)], src_i)
        pltpu.sync_copy(dst_hbm.at[pl.ds(wid * nch, nch)], dst_i)

        @pl.loop(0, npad // 16)
        def _(i):
            cnts_v[pl.ds(i * 16, 16)] = jnp.zeros((16,), jnp.float32)
            cntd_v[pl.ds(i * 16, 16)] = jnp.zeros((16,), jnp.float32)

        ones16 = jnp.ones((16,), jnp.float32)

        @pl.loop(0, nch)
        def _(c):
            for j in range(CHUNK // 16):
                plsc.addupdate_scatter(
                    cnts_v, [src_i[c, pl.ds(j * 16, 16)]], ones16)
                plsc.addupdate_scatter(
                    cntd_v, [dst_i[c, pl.ds(j * 16, 16)]], ones16)

        pltpu.sync_copy(cnts_v, outs_hbm.at[pl.ds(wid * npad, npad)])
        pltpu.sync_copy(cntd_v, outd_hbm.at[pl.ds(wid * npad, npad)])

    return k


IW = 8


@functools.lru_cache(maxsize=None)
def _make_segsum_kernel(n, e, d):
    nch = _edge_chunks(e)
    npad = _pad_rows(n)
    rows_per_tile = npad // NS
    assert nch % IW == 0 and IW % 2 == 0
    nwin = nch // IW

    @functools.partial(
        pl.kernel,
        mesh=_mesh(),
        out_type=jax.ShapeDtypeStruct((NC, npad, d), jnp.float32),
        scratch_types=[
            pltpu.VMEM((2, IW, CHUNK), jnp.int32),
            pltpu.VMEM((2, IW, CHUNK), jnp.int32),
            pltpu.VMEM((CHUNK, d), jnp.float32),
            pltpu.VMEM((CHUNK, d), jnp.float32),
            pltpu.VMEM_SHARED((npad, d), jnp.float32),
            pltpu.SemaphoreType.DMA((2,)),
            pltpu.SemaphoreType.DMA((2,)),
        ],
    )
    def k(z_hbm, src_hbm, dst_hbm, out_hbm,
          srcw, dstw, bufa, bufb, acc_sh, gsems, isems):
        cid = lax.axis_index("c")
        sid = lax.axis_index("s")
        wid = sid * NC + cid
        cb = wid * nch
        buf = [bufa, bufb]

        def idx_load(win, slot, sem_slot):
            pltpu.async_copy(src_hbm.at[pl.ds(cb + win * IW, IW)],
                             srcw.at[slot], isems.at[sem_slot])
            pltpu.async_copy(dst_hbm.at[pl.ds(cb + win * IW, IW)],
                             dstw.at[slot], isems.at[1 - sem_slot])

        def idx_wait():
            pltpu.make_async_copy(src_hbm.at[pl.ds(0, IW)], srcw.at[0],
                                  isems.at[0]).wait()
            pltpu.make_async_copy(src_hbm.at[pl.ds(0, IW)], srcw.at[0],
                                  isems.at[1]).wait()

        def gather(idx_row, q):
            pltpu.async_copy(z_hbm.at[idx_row], buf[q], gsems.at[q])

        def gather_wait(q):
            pltpu.make_async_copy(z_hbm.at[srcw.at[0, 0]], buf[q],
                                  gsems.at[q]).wait()

        idx_load(0, 0, 0)

        _zero_rows(bufa, CHUNK, d)
        base = sid * rows_per_tile
        _copy_rows_spmem(bufa, acc_sh, base, rows_per_tile)
        plsc.subcore_barrier()

        idx_wait()
        gather(srcw.at[0, 0], 0)

        @pl.loop(0, nwin)
        def _(w):
            p = jnp.bitwise_and(w, 1)

            @pl.when(w < nwin - 1)
            def _():
                idx_load(w + 1, 1 - p, 0)

            for c in range(IW):
                q = c & 1
                gather_wait(q)
                if c < IW - 1:
                    gather(srcw.at[p, c + 1], 1 - q)
                else:
                    @pl.when(w < nwin - 1)
                    def _():
                        idx_wait()
                        gather(srcw.at[1 - p, 0], 1 - q)
                pltpu.sync_copy(buf[q], acc_sh.at[dstw.at[p, c]], add=True)

        plsc.subcore_barrier()
        _copy_out_spmem(acc_sh, out_hbm, cid, base, rows_per_tile)

    return k


def _norm_col(parts_ref):
    deg = jnp.sum(parts_ref[...], axis=1, keepdims=True)
    return lax.rsqrt(jnp.maximum(deg, 1.0))


def _tc_pre(x, degs_t, w1t):
    n, d = x.shape
    r = 1000

    def body(x_ref, degs_ref, w_ref, z_ref):
        norm = _norm_col(degs_ref)
        h = x_ref[...] * norm
        z_ref[...] = jnp.dot(h, w_ref[...], preferred_element_type=jnp.float32)

    return pl.pallas_call(
        body,
        grid=(n // r,),
        in_specs=[
            pl.BlockSpec((r, d), lambda i: (i, 0)),
            pl.BlockSpec((r, NW), lambda i: (i, 0)),
            pl.BlockSpec((d, d), lambda i: (0, 0)),
        ],
        out_specs=pl.BlockSpec((r, d), lambda i: (i, 0)),
        out_shape=jax.ShapeDtypeStruct((n, d), jnp.float32),
    )(x, degs_t, w1t)


def _tc_mid(agg_parts, degd_t, degs_t, b1, gamma, beta, a, w2t, n):
    d = agg_parts.shape[-1]
    r = 1000

    def body(agg_ref, degd_ref, degs_ref, b_ref, g_ref, bt_ref, a_ref, w_ref,
             z_ref):
        nd = _norm_col(degd_ref)
        h = (agg_ref[0] + agg_ref[1] + b_ref[...]) * nd
        mean = jnp.mean(h, axis=1, keepdims=True)
        var = jnp.mean((h - mean) ** 2, axis=1, keepdims=True)
        hn = (h - mean) * lax.rsqrt(var + 1e-5) * g_ref[...] + bt_ref[...]
        hp = jnp.where(hn > 0, hn, a_ref[0, 0] * hn)
        ns = _norm_col(degs_ref)
        z_ref[...] = jnp.dot(hp * ns, w_ref[...],
                             preferred_element_type=jnp.float32)

    return pl.pallas_call(
        body,
        grid=(n // r,),
        in_specs=[
            pl.BlockSpec((NC, r, d), lambda i: (0, i, 0)),
            pl.BlockSpec((r, NW), lambda i: (i, 0)),
            pl.BlockSpec((r, NW), lambda i: (i, 0)),
            pl.BlockSpec((1, d), lambda i: (0, 0)),
            pl.BlockSpec((1, d), lambda i: (0, 0)),
            pl.BlockSpec((1, d), lambda i: (0, 0)),
            pl.BlockSpec((1, 1), lambda i: (0, 0), memory_space=pltpu.SMEM),
            pl.BlockSpec((d, d), lambda i: (0, 0)),
        ],
        out_specs=pl.BlockSpec((r, d), lambda i: (i, 0)),
        out_shape=jax.ShapeDtypeStruct((n, d), jnp.float32),
    )(agg_parts, degd_t, degs_t, b1, gamma, beta, a, w2t)


def _tc_post(agg_parts, degd_t, b2, n):
    d = agg_parts.shape[-1]
    r = 1000

    def body(agg_ref, degd_ref, b_ref, o_ref):
        nd = _norm_col(degd_ref)
        o_ref[...] = (agg_ref[0] + agg_ref[1] + b_ref[...]) * nd

    return pl.pallas_call(
        body,
        grid=(n // r,),
        in_specs=[
            pl.BlockSpec((NC, r, d), lambda i: (0, i, 0)),
            pl.BlockSpec((r, NW), lambda i: (i, 0)),
            pl.BlockSpec((1, d), lambda i: (0, 0)),
        ],
        out_specs=pl.BlockSpec((r, d), lambda i: (i, 0)),
        out_shape=jax.ShapeDtypeStruct((n, d), jnp.float32),
    )(agg_parts, degd_t, b2)


def kernel(x, edge_index, W1, b1, ln_gamma, ln_beta, prelu_a, W2, b2):
    n, d = x.shape
    e = edge_index.shape[1]
    src = edge_index[0]
    dst = edge_index[1]

    nch = _edge_chunks(e)
    pad = NW * nch * CHUNK - e
    dst_p = jnp.concatenate([dst, jnp.full((pad,), n, jnp.int32)])
    dst_p = dst_p.reshape(NW * nch, CHUNK)
    src_deg = jnp.concatenate([src, jnp.full((pad,), n, jnp.int32)])
    src_deg = src_deg.reshape(NW * nch, CHUNK)
    src_seg = jnp.concatenate([src, jnp.zeros((pad,), jnp.int32)])
    src_seg = src_seg.reshape(NW * nch, CHUNK)

    degs_parts, degd_parts = _make_degree_kernel(n, e)(src_deg, dst_p)
    npad = _pad_rows(n)
    degs_t = degs_parts.reshape(NW, npad)[:, :n].T
    degd_t = degd_parts.reshape(NW, npad)[:, :n].T

    segsum = _make_segsum_kernel(n, e, d)

    z1 = _tc_pre(x, degs_t, W1.T)
    agg1 = segsum(z1, src_seg, dst_p)
    z2 = _tc_mid(agg1, degd_t, degs_t,
                 b1.reshape(1, d), ln_gamma.reshape(1, d),
                 ln_beta.reshape(1, d), prelu_a.reshape(1, 1), W2.T, n)
    agg2 = segsum(z2, src_seg, dst_p)
    return _tc_post(agg2, degd_t, b2.reshape(1, d), n)
